# baseline (speedup 1.0000x reference)
import numpy as np

import jax
import jax.numpy as jnp
from jax.experimental import pallas as pl
from jax.experimental.pallas import tpu as pltpu

_D = 1024
_B = 8
_K = max(1, int(0.1 * _D * _D))
_CHUNK = 128
_C = _D // _CHUNK
_LPS = 4
_P = _B + 2
_MASK31 = 0x7FFFFFFF


def _to_key(x):
    bits = jax.lax.bitcast_convert_type(x, jnp.int32)
    return jnp.where(bits < 0, bits ^ _MASK31, bits)


def _from_key(key):
    bits = jnp.where(key < 0, key ^ _MASK31, key)
    return jax.lax.bitcast_convert_type(bits, jnp.float32)


def _body(env_ref, temp_ref, base_ref, delta_ref,
          a_ref, logits_ref, soft_ref, keys3_ref, base_vmem,
          state_ref, kth_ref):
    p = pl.program_id(0)
    i = pl.program_id(1)

    @pl.when(p < _B)
    def _build():
        row = i * _CHUNK

        @pl.when(p == 0)
        def _fill_cache():
            base_vmem[pl.ds(row, _CHUNK), :] = base_ref[...]

        x = base_vmem[pl.ds(row, _CHUNK), :] + delta_ref[0]
        logits_ref[0] = x
        soft_ref[0] = jax.nn.sigmoid(x * (1.0 / temp_ref[0]))
        bufp = jax.lax.rem(p, 3)
        keys3_ref[bufp, pl.ds(row, _CHUNK), :] = _to_key(x)

    @pl.when((p >= 1) & (p <= _B))
    def _count():
        bufq = jax.lax.rem(p - 1, 3)
        kb = keys3_ref.at[bufq]

        @pl.when(i == 0)
        def _init():
            state_ref[0] = np.int32(-2147483648)

        lo = state_ref[0]
        kk = np.int32(_K)
        nslc = 32
        rows = _D // nslc
        slices = [kb[j * rows:(j + 1) * rows, :] for j in range(nslc)]
        base_level = i * _LPS
        for l in range(_LPS):
            shift = 31 - (base_level + l)
            delta = np.int32(1) << shift
            mid = lo + delta
            parts = [jnp.sum((sl >= mid).astype(jnp.int32)) for sl in slices]
            cnt = sum(parts)
            lo = jnp.where(cnt >= kk, mid, lo)
        state_ref[0] = lo
        kth_ref[jnp.clip(p - 1, 0, _B - 1)] = lo

    @pl.when(p >= 2)
    def _emit():
        bufr = jax.lax.rem(p - 2, 3)
        kth = kth_ref[jnp.clip(p - 2, 0, _B - 1)]
        key = keys3_ref[bufr, pl.ds(i * _CHUNK, _CHUNK), :]
        x = _from_key(key)
        a_ref[0] = jnp.where(key >= kth, jax.nn.sigmoid(x), 0.0)


def kernel(z_s, env_idx, A_base, A_deltas, temperature):
    del z_s
    b, d = _B, _D
    env = env_idx.astype(jnp.int32)
    temp = jnp.asarray(temperature, jnp.float32).reshape(1)

    grid_spec = pltpu.PrefetchScalarGridSpec(
        num_scalar_prefetch=1,
        grid=(_P, _C),
        in_specs=[
            pl.BlockSpec(memory_space=pltpu.MemorySpace.SMEM),
            pl.BlockSpec(
                (_CHUNK, d),
                lambda p, i, e: (jnp.where(p == 0, i, _C - 1), 0)),
            pl.BlockSpec(
                (1, _CHUNK, d),
                lambda p, i, e: (e[jnp.clip(p, 0, _B - 1)],
                                 jnp.where(p < _B, i, _C - 1), 0)),
        ],
        out_specs=[
            pl.BlockSpec(
                (1, _CHUNK, d),
                lambda p, i, e: (jnp.clip(p - 2, 0, _B - 1),
                                 jnp.where(p >= 2, i, 0), 0)),
            pl.BlockSpec(
                (1, _CHUNK, d),
                lambda p, i, e: (jnp.clip(p, 0, _B - 1),
                                 jnp.where(p < _B, i, _C - 1), 0)),
            pl.BlockSpec(
                (1, _CHUNK, d),
                lambda p, i, e: (jnp.clip(p, 0, _B - 1),
                                 jnp.where(p < _B, i, _C - 1), 0)),
        ],
        scratch_shapes=[
            pltpu.MemorySpace.VMEM((3, d, d), jnp.int32),
            pltpu.MemorySpace.VMEM((d, d), jnp.float32),
            pltpu.MemorySpace.SMEM((1,), jnp.int32),
            pltpu.MemorySpace.SMEM((_B,), jnp.int32),
        ],
    )
    out_shape = [
        jax.ShapeDtypeStruct((b, d, d), jnp.float32),
        jax.ShapeDtypeStruct((b, d, d), jnp.float32),
        jax.ShapeDtypeStruct((b, d, d), jnp.float32),
    ]
    a, logits, soft = pl.pallas_call(
        _body,
        grid_spec=grid_spec,
        out_shape=out_shape,
    )(env, temp, A_base, A_deltas)
    return (a, logits, soft)


# speedup vs baseline: 1.0172x; 1.0172x over previous
import numpy as np

import jax
import jax.numpy as jnp
from jax.experimental import pallas as pl
from jax.experimental.pallas import tpu as pltpu

_D = 1024
_B = 8
_K = max(1, int(0.1 * _D * _D))
_CHUNK = 128
_C = _D // _CHUNK
_LPS = 4
_P = _B + 2
_MASK31 = 0x7FFFFFFF


def _to_key(x):
    bits = jax.lax.bitcast_convert_type(x, jnp.int32)
    return jnp.where(bits < 0, bits ^ _MASK31, bits)


def _from_key(key):
    bits = jnp.where(key < 0, key ^ _MASK31, key)
    return jax.lax.bitcast_convert_type(bits, jnp.float32)


def _body(env_ref, temp_ref, base_ref, delta_ref,
          a_ref, logits_ref, soft_ref, keys3_ref, base_vmem,
          state_ref, kth_ref):
    p = pl.program_id(0)
    i = pl.program_id(1)

    @pl.when(p < _B)
    def _build():
        row = i * _CHUNK

        @pl.when(p == 0)
        def _fill_cache():
            base_vmem[pl.ds(row, _CHUNK), :] = base_ref[...]

        x = base_vmem[pl.ds(row, _CHUNK), :] + delta_ref[0]
        logits_ref[0] = x
        soft_ref[0] = jax.nn.sigmoid(x * (1.0 / temp_ref[0]))
        bufp = jax.lax.rem(p, 3)
        keys3_ref[bufp, pl.ds(row, _CHUNK), :] = _to_key(x)

    @pl.when((p >= 1) & (p <= _B))
    def _count():
        bufq = jax.lax.rem(p - 1, 3)
        kb = keys3_ref.at[bufq]

        @pl.when(i == 0)
        def _init():
            state_ref[0] = np.int32(-2147483648)

        lo = state_ref[0]
        kk = np.int32(_K)
        nslc = 16
        rows = _D // nslc
        slices = [kb[j * rows:(j + 1) * rows, :] for j in range(nslc)]
        base_level = i * _LPS
        for l in range(_LPS):
            shift = 31 - (base_level + l)
            delta = np.int32(1) << shift
            mid = lo + delta
            parts = [jnp.sum((sl >= mid).astype(jnp.int32)) for sl in slices]
            cnt = sum(parts)
            lo = jnp.where(cnt >= kk, mid, lo)
        state_ref[0] = lo
        kth_ref[jnp.clip(p - 1, 0, _B - 1)] = lo

    @pl.when(p >= 2)
    def _emit():
        bufr = jax.lax.rem(p - 2, 3)
        kth = kth_ref[jnp.clip(p - 2, 0, _B - 1)]
        key = keys3_ref[bufr, pl.ds(i * _CHUNK, _CHUNK), :]
        x = _from_key(key)
        a_ref[0] = jnp.where(key >= kth, jax.nn.sigmoid(x), 0.0)


def kernel(z_s, env_idx, A_base, A_deltas, temperature):
    del z_s
    b, d = _B, _D
    env = env_idx.astype(jnp.int32)
    temp = jnp.asarray(temperature, jnp.float32).reshape(1)

    grid_spec = pltpu.PrefetchScalarGridSpec(
        num_scalar_prefetch=1,
        grid=(_P, _C),
        in_specs=[
            pl.BlockSpec(memory_space=pltpu.MemorySpace.SMEM),
            pl.BlockSpec(
                (_CHUNK, d),
                lambda p, i, e: (jnp.where(p == 0, i, _C - 1), 0)),
            pl.BlockSpec(
                (1, _CHUNK, d),
                lambda p, i, e: (e[jnp.clip(p, 0, _B - 1)],
                                 jnp.where(p < _B, i, _C - 1), 0)),
        ],
        out_specs=[
            pl.BlockSpec(
                (1, _CHUNK, d),
                lambda p, i, e: (jnp.clip(p - 2, 0, _B - 1),
                                 jnp.where(p >= 2, i, 0), 0)),
            pl.BlockSpec(
                (1, _CHUNK, d),
                lambda p, i, e: (jnp.clip(p, 0, _B - 1),
                                 jnp.where(p < _B, i, _C - 1), 0)),
            pl.BlockSpec(
                (1, _CHUNK, d),
                lambda p, i, e: (jnp.clip(p, 0, _B - 1),
                                 jnp.where(p < _B, i, _C - 1), 0)),
        ],
        scratch_shapes=[
            pltpu.MemorySpace.VMEM((3, d, d), jnp.int32),
            pltpu.MemorySpace.VMEM((d, d), jnp.float32),
            pltpu.MemorySpace.SMEM((1,), jnp.int32),
            pltpu.MemorySpace.SMEM((_B,), jnp.int32),
        ],
    )
    out_shape = [
        jax.ShapeDtypeStruct((b, d, d), jnp.float32),
        jax.ShapeDtypeStruct((b, d, d), jnp.float32),
        jax.ShapeDtypeStruct((b, d, d), jnp.float32),
    ]
    a, logits, soft = pl.pallas_call(
        _body,
        grid_spec=grid_spec,
        out_shape=out_shape,
    )(env, temp, A_base, A_deltas)
    return (a, logits, soft)
